# baseline (speedup 1.0000x reference)
import functools

import jax
import jax.numpy as jnp
from jax import lax
from jax.experimental import pallas as pl
from jax.experimental.pallas import tpu as pltpu
from jax.experimental.pallas import tpu_sc as plsc

N = 10000
E = 320000
F_IN = 128
C = 64

NC = 2
NS = 16
NW = NC * NS
EPT = 10240
EPAD = EPT * NW
CHUNK = 128
NCHUNK = EPT // CHUNK
KF = 4

NPAD = 10240
ROWS_PT = NPAD // NS
RS = NPAD // NS

@functools.cache
def _make_deg_kernel():
    mesh = plsc.VectorSubcoreMesh(
        core_axis_name="c", subcore_axis_name="s", num_cores=NC, num_subcores=NS
    )
    return pl.kernel(
        _deg_body,
        out_type=jax.ShapeDtypeStruct((NC, NPAD, 16), jnp.float32),
        mesh=mesh,
        compiler_params=pltpu.CompilerParams(use_tc_tiling_on_sc=False),
        scratch_types=[
            pltpu.VMEM_SHARED((NPAD, 16), jnp.float32),
            pltpu.VMEM((NCHUNK, CHUNK), jnp.int32),
            pltpu.VMEM((CHUNK, 16), jnp.float32),
            pltpu.SemaphoreType.DMA,
        ],
    )


def _deg_body(dst3_hbm, zeros_hbm, ones_hbm, out_hbm, deg_sh, didx, ones, ssem):
    cid = lax.axis_index("c")
    sid = lax.axis_index("s")
    wid = cid * NS + sid

    pltpu.sync_copy(ones_hbm, ones)
    pltpu.sync_copy(dst3_hbm.at[wid], didx)
    pltpu.sync_copy(
        zeros_hbm.at[pl.ds(sid * ROWS_PT, ROWS_PT)],
        deg_sh.at[pl.ds(sid * ROWS_PT, ROWS_PT)],
    )
    plsc.subcore_barrier()

    def round_body(r, _):
        ds = []
        for b in range(KF):
            c = r * KF + b
            ds.append(pltpu.async_copy(ones, deg_sh.at[didx.at[c]], ssem,
                                       add=True))
        for d in ds:
            d.wait()
        return 0

    lax.fori_loop(0, NCHUNK // KF, round_body, 0)
    plsc.subcore_barrier()
    pltpu.sync_copy(
        deg_sh.at[pl.ds(sid * ROWS_PT, ROWS_PT)],
        out_hbm.at[cid, pl.ds(sid * ROWS_PT, ROWS_PT)],
    )


@functools.cache
def _make_hop_kernel():
    mesh = plsc.VectorSubcoreMesh(
        core_axis_name="c", subcore_axis_name="s", num_cores=NC, num_subcores=NS
    )
    return pl.kernel(
        _hop_body,
        out_type=jax.ShapeDtypeStruct((NC, NPAD, C), jnp.float32),
        mesh=mesh,
        compiler_params=pltpu.CompilerParams(use_tc_tiling_on_sc=False),
        scratch_types=[
            pltpu.VMEM_SHARED((NPAD, C), jnp.float32),
            pltpu.VMEM((NCHUNK, CHUNK), jnp.int32),
            pltpu.VMEM((NCHUNK, CHUNK), jnp.int32),
            pltpu.VMEM((KF * CHUNK, C), jnp.float32),
            pltpu.VMEM((KF * CHUNK, C), jnp.float32),
            pltpu.SemaphoreType.DMA,
            pltpu.SemaphoreType.DMA,
            pltpu.SemaphoreType.DMA,
            pltpu.SemaphoreType.DMA,
        ],
    )


NROUND = NCHUNK // KF


def _hop_body(u_hbm, src3_hbm, dst3_hbm, out_hbm, acc_sh, sidx, didx,
              rows0, rows1, gsem0, gsem1, ssem0, ssem1):
    cid = lax.axis_index("c")
    sid = lax.axis_index("s")
    wid = cid * NS + sid

    rows = (rows0, rows1)
    gsem = (gsem0, gsem1)
    ssem = (ssem0, ssem1)

    def fire_gathers(r, x):
        for b in range(KF):
            pltpu.async_copy(u_hbm.at[sidx.at[r * KF + b]],
                             rows[x].at[pl.ds(b * CHUNK, CHUNK)], gsem[x])

    def drain_gathers(x):
        for b in range(KF):
            pltpu.make_async_copy(u_hbm.at[sidx.at[0]],
                                  rows[x].at[pl.ds(b * CHUNK, CHUNK)],
                                  gsem[x]).wait()

    def fire_scatters(r, x):
        for b in range(KF):
            pltpu.async_copy(rows[x].at[pl.ds(b * CHUNK, CHUNK)],
                             acc_sh.at[didx.at[r * KF + b]], ssem[x],
                             add=True)

    def drain_scatters(x):
        for b in range(KF):
            pltpu.make_async_copy(rows[x].at[pl.ds(b * CHUNK, CHUNK)],
                                  acc_sh.at[didx.at[0]], ssem[x]).wait()

    pltpu.sync_copy(u_hbm.at[pl.ds(sid * RS, RS)], acc_sh.at[pl.ds(sid * RS, RS)])
    pltpu.sync_copy(src3_hbm.at[wid], sidx)
    pltpu.sync_copy(dst3_hbm.at[wid], didx)
    plsc.subcore_barrier()

    fire_gathers(0, 0)
    drain_gathers(0)
    fire_scatters(0, 0)
    fire_gathers(1, 1)

    def pair_body(i, _):
        for (roff, x) in ((1, 1), (2, 0)):
            r = 2 * i + roff
            drain_gathers(x)
            fire_scatters(r, x)
            drain_scatters(1 - x)
            fire_gathers(r + 1, 1 - x)
        return 0

    lax.fori_loop(0, (NROUND - 2) // 2, pair_body, 0)
    drain_gathers(1)
    fire_scatters(NROUND - 1, 1)
    drain_scatters(0)
    drain_scatters(1)
    plsc.subcore_barrier()
    pltpu.sync_copy(
        acc_sh.at[pl.ds(sid * RS, RS)],
        out_hbm.at[cid, pl.ds(sid * RS, RS)],
    )


CHUNK2 = 80
NCHUNK2 = EPT // CHUNK2
KF2 = 2
NROUND2 = NCHUNK2 // KF2
RC = 32
NRC = RS // RC


@functools.cache
def _make_hop2_kernel():
    mesh = plsc.VectorSubcoreMesh(
        core_axis_name="c", subcore_axis_name="s", num_cores=NC, num_subcores=NS
    )
    return pl.kernel(
        _hop2_body,
        out_type=jax.ShapeDtypeStruct((NC, NPAD, C), jnp.float32),
        mesh=mesh,
        compiler_params=pltpu.CompilerParams(use_tc_tiling_on_sc=False),
        scratch_types=[
            pltpu.VMEM_SHARED((NPAD, C), jnp.float32),
            pltpu.VMEM_SHARED((NPAD, C), jnp.float32),
            pltpu.VMEM((NCHUNK2, CHUNK2), jnp.int32),
            pltpu.VMEM((NCHUNK2, CHUNK2), jnp.int32),
            pltpu.VMEM((KF2 * CHUNK2, C), jnp.float32),
            pltpu.VMEM((KF2 * CHUNK2, C), jnp.float32),
            pltpu.VMEM((RC, C), jnp.float32),
            pltpu.VMEM((RC, C), jnp.float32),
            pltpu.VMEM((RC, C), jnp.float32),
            pltpu.VMEM((RS,), jnp.float32),
            pltpu.SemaphoreType.DMA,
            pltpu.SemaphoreType.DMA,
            pltpu.SemaphoreType.DMA,
            pltpu.SemaphoreType.DMA,
        ],
    )


def _hop2_body(u_hbm, winv_hbm, p_hbm, src3_hbm, dst3_hbm, out_hbm,
               w_sh, acc_sh, sidx, didx, rows0, rows1, ab, bb, cb, winvb,
               gsem0, gsem1, ssem0, ssem1):
    cid = lax.axis_index("c")
    sid = lax.axis_index("s")
    wid = cid * NS + sid
    base = sid * RS

    rows = (rows0, rows1)
    gsem = (gsem0, gsem1)
    ssem = (ssem0, ssem1)

    def fire_gathers(r, x):
        for b in range(KF2):
            pltpu.async_copy(w_sh.at[sidx.at[r * KF2 + b]],
                             rows[x].at[pl.ds(b * CHUNK2, CHUNK2)], gsem[x])

    def drain_gathers(x):
        for b in range(KF2):
            pltpu.make_async_copy(w_sh.at[sidx.at[0]],
                                  rows[x].at[pl.ds(b * CHUNK2, CHUNK2)],
                                  gsem[x]).wait()

    def fire_scatters(r, x):
        for b in range(KF2):
            pltpu.async_copy(rows[x].at[pl.ds(b * CHUNK2, CHUNK2)],
                             acc_sh.at[didx.at[r * KF2 + b]], ssem[x],
                             add=True)

    def drain_scatters(x):
        for b in range(KF2):
            pltpu.make_async_copy(rows[x].at[pl.ds(b * CHUNK2, CHUNK2)],
                                  acc_sh.at[didx.at[0]], ssem[x]).wait()

    pltpu.sync_copy(src3_hbm.at[wid], sidx)
    pltpu.sync_copy(dst3_hbm.at[wid], didx)
    pltpu.sync_copy(winv_hbm.at[pl.ds(base, RS)], winvb)

    def phase_b(rc, _):
        lo = base + rc * RC
        pltpu.sync_copy(p_hbm.at[0, pl.ds(lo, RC)], ab)
        pltpu.sync_copy(p_hbm.at[1, pl.ds(lo, RC)], bb)
        pltpu.sync_copy(u_hbm.at[pl.ds(lo, RC)], cb)

        def comb(g, _):
            wvv = winvb[pl.ds(rc * RC + g * 16, 16)]
            for r16 in range(16):
                r = g * 16 + r16
                for j in range(C // 16):
                    sl = pl.ds(j * 16, 16)
                    ab[r, sl] = (ab[r, sl] + bb[r, sl] - cb[r, sl]) * wvv[r16]
            return 0

        lax.fori_loop(0, RC // 16, comb, 0)
        pltpu.sync_copy(ab, w_sh.at[pl.ds(lo, RC)])
        pltpu.sync_copy(ab, acc_sh.at[pl.ds(lo, RC)])
        return 0

    lax.fori_loop(0, NRC, phase_b, 0)
    plsc.subcore_barrier()

    fire_gathers(0, 0)
    drain_gathers(0)
    fire_scatters(0, 0)
    fire_gathers(1, 1)

    def pair_body(i, _):
        for (roff, x) in ((1, 1), (2, 0)):
            r = 2 * i + roff
            drain_gathers(x)
            fire_scatters(r, x)
            drain_scatters(1 - x)
            fire_gathers(r + 1, 1 - x)
        return 0

    lax.fori_loop(0, (NROUND2 - 2) // 2, pair_body, 0)
    drain_gathers(1)
    fire_scatters(NROUND2 - 1, 1)
    drain_scatters(0)
    drain_scatters(1)
    plsc.subcore_barrier()

    cmul = 1.0 - lax.convert_element_type(cid, jnp.float32)

    def phase_d(rc, _):
        lo = base + rc * RC
        pltpu.sync_copy(acc_sh.at[pl.ds(lo, RC)], ab)
        pltpu.sync_copy(w_sh.at[pl.ds(lo, RC)], bb)

        def outc(r, _):
            for j in range(C // 16):
                sl = pl.ds(j * 16, 16)
                ab[r, sl] = ab[r, sl] - cmul * bb[r, sl]
            return 0

        lax.fori_loop(0, RC, outc, 0)
        pltpu.sync_copy(ab, out_hbm.at[cid, pl.ds(lo, RC)])
        return 0

    lax.fori_loop(0, NRC, phase_d, 0)


def _project_body(x_ref, w_ref, degp_ref, u_ref, dinv_ref, winv_ref):
    deg = degp_ref[0, :, 0:1] + degp_ref[1, :, 0:1] + 1.0
    dinv = lax.rsqrt(deg)
    winv = 1.0 / deg
    z = lax.dot_general(
        x_ref[...], w_ref[...],
        dimension_numbers=(((1,), (1,)), ((), ())),
        preferred_element_type=jnp.float32,
        precision=lax.Precision.HIGHEST,
    )
    zp = jnp.concatenate([z, jnp.zeros((NPAD - N, C), jnp.float32)], axis=0)
    u_ref[...] = zp * dinv
    dinv_ref[...] = dinv
    winv_ref[...] = winv


_project = pl.pallas_call(
    _project_body,
    out_shape=(
        jax.ShapeDtypeStruct((NPAD, C), jnp.float32),
        jax.ShapeDtypeStruct((NPAD, 1), jnp.float32),
        jax.ShapeDtypeStruct((NPAD, 1), jnp.float32),
    ),
)


def _combine_body(p_ref, u_ref, winv_ref, w_ref):
    w_ref[...] = (p_ref[0] + p_ref[1] - u_ref[...]) * winv_ref[...]


_combine = pl.pallas_call(
    _combine_body,
    out_shape=jax.ShapeDtypeStruct((NPAD, C), jnp.float32),
)


def _final_body(q_ref, dinv_ref, b_ref, o_ref):
    h = (q_ref[0] + q_ref[1]) * dinv_ref[...]
    o = h[:N] + b_ref[...]
    m = jnp.max(o, axis=1, keepdims=True)
    s = jnp.sum(jnp.exp(o - m), axis=1, keepdims=True)
    o_ref[...] = (o - m) - jnp.log(s)


_final = pl.pallas_call(
    _final_body,
    out_shape=jax.ShapeDtypeStruct((N, C), jnp.float32),
)


def kernel(x, edge_index, W, b):
    pad = N + jax.lax.rem(jnp.arange(EPAD - E, dtype=jnp.int32),
                          jnp.int32(NPAD - N))
    src3 = jnp.reshape(jnp.concatenate([edge_index[0], pad]), (NW, NCHUNK, CHUNK))
    dst3 = jnp.reshape(jnp.concatenate([edge_index[1], pad]), (NW, NCHUNK, CHUNK))
    zeros16 = jnp.zeros((NPAD, 16), jnp.float32)
    ones16 = jnp.ones((CHUNK, 16), jnp.float32)
    src3b = jnp.reshape(src3, (NW, NCHUNK2, CHUNK2))
    dst3b = jnp.reshape(dst3, (NW, NCHUNK2, CHUNK2))
    deg_kernel = _make_deg_kernel()
    hop_kernel = _make_hop_kernel()
    hop2_kernel = _make_hop2_kernel()
    degp = deg_kernel(dst3, zeros16, ones16)
    u, dinv, winv = _project(x, W, degp)
    p = hop_kernel(u, src3, dst3)
    q = hop2_kernel(u, jnp.reshape(winv, (NPAD,)), p, src3b, dst3b)
    return _final(q, dinv, jnp.reshape(b, (1, C)))


# speedup vs baseline: 1.1776x; 1.1776x over previous
import functools

import jax
import jax.numpy as jnp
from jax import lax
from jax.experimental import pallas as pl
from jax.experimental.pallas import tpu as pltpu
from jax.experimental.pallas import tpu_sc as plsc

N = 10000
E = 320000
F_IN = 128
C = 64

NC = 2
NS = 16
NW = NC * NS
EPT = 10240
EPAD = EPT * NW
CHUNK = 128
NCHUNK = EPT // CHUNK
KF = 4

NPAD = 10240
ROWS_PT = NPAD // NS
RS = NPAD // NS

@functools.cache
def _make_deg_kernel():
    mesh = plsc.VectorSubcoreMesh(
        core_axis_name="c", subcore_axis_name="s", num_cores=NC, num_subcores=NS
    )
    return pl.kernel(
        _deg_body,
        out_type=jax.ShapeDtypeStruct((NC, NPAD, 16), jnp.float32),
        mesh=mesh,
        compiler_params=pltpu.CompilerParams(use_tc_tiling_on_sc=False),
        scratch_types=[
            pltpu.VMEM_SHARED((NPAD, 16), jnp.float32),
            pltpu.VMEM((NCHUNK, CHUNK), jnp.int32),
            pltpu.VMEM((CHUNK, 16), jnp.float32),
            pltpu.SemaphoreType.DMA,
        ],
    )


def _deg_body(dst3_hbm, zeros_hbm, ones_hbm, out_hbm, deg_sh, didx, ones, ssem):
    cid = lax.axis_index("c")
    sid = lax.axis_index("s")
    wid = cid * NS + sid

    pltpu.sync_copy(ones_hbm, ones)
    pltpu.sync_copy(dst3_hbm.at[wid], didx)
    pltpu.sync_copy(
        zeros_hbm.at[pl.ds(sid * ROWS_PT, ROWS_PT)],
        deg_sh.at[pl.ds(sid * ROWS_PT, ROWS_PT)],
    )
    plsc.subcore_barrier()

    def fire_round(r):
        for b in range(KF):
            pltpu.async_copy(ones, deg_sh.at[didx.at[r * KF + b]], ssem,
                             add=True)

    def drain_round():
        for b in range(KF):
            pltpu.make_async_copy(ones, deg_sh.at[didx.at[0]], ssem).wait()

    fire_round(0)

    def round_body(r, _):
        fire_round(r + 1)
        drain_round()
        return 0

    lax.fori_loop(0, NCHUNK // KF - 1, round_body, 0)
    drain_round()
    plsc.subcore_barrier()
    pltpu.sync_copy(
        deg_sh.at[pl.ds(sid * ROWS_PT, ROWS_PT)],
        out_hbm.at[cid, pl.ds(sid * ROWS_PT, ROWS_PT)],
    )


@functools.cache
def _make_hop_kernel():
    mesh = plsc.VectorSubcoreMesh(
        core_axis_name="c", subcore_axis_name="s", num_cores=NC, num_subcores=NS
    )
    return pl.kernel(
        _hop_body,
        out_type=jax.ShapeDtypeStruct((NC, NPAD, C), jnp.float32),
        mesh=mesh,
        compiler_params=pltpu.CompilerParams(use_tc_tiling_on_sc=False),
        scratch_types=[
            pltpu.VMEM_SHARED((NPAD, C), jnp.float32),
            pltpu.VMEM((NCHUNK, CHUNK), jnp.int32),
            pltpu.VMEM((NCHUNK, CHUNK), jnp.int32),
            pltpu.VMEM((KF * CHUNK, C), jnp.float32),
            pltpu.VMEM((KF * CHUNK, C), jnp.float32),
            pltpu.SemaphoreType.DMA,
            pltpu.SemaphoreType.DMA,
            pltpu.SemaphoreType.DMA,
            pltpu.SemaphoreType.DMA,
        ],
    )


NROUND = NCHUNK // KF


def _hop_body(u_hbm, src3_hbm, dst3_hbm, out_hbm, acc_sh, sidx, didx,
              rows0, rows1, gsem0, gsem1, ssem0, ssem1):
    cid = lax.axis_index("c")
    sid = lax.axis_index("s")
    wid = cid * NS + sid

    rows = (rows0, rows1)
    gsem = (gsem0, gsem1)
    ssem = (ssem0, ssem1)

    def fire_gathers(r, x):
        for b in range(KF):
            pltpu.async_copy(u_hbm.at[sidx.at[r * KF + b]],
                             rows[x].at[pl.ds(b * CHUNK, CHUNK)], gsem[x])

    def drain_gathers(x):
        for b in range(KF):
            pltpu.make_async_copy(u_hbm.at[sidx.at[0]],
                                  rows[x].at[pl.ds(b * CHUNK, CHUNK)],
                                  gsem[x]).wait()

    def fire_scatters(r, x):
        for b in range(KF):
            pltpu.async_copy(rows[x].at[pl.ds(b * CHUNK, CHUNK)],
                             acc_sh.at[didx.at[r * KF + b]], ssem[x],
                             add=True)

    def drain_scatters(x):
        for b in range(KF):
            pltpu.make_async_copy(rows[x].at[pl.ds(b * CHUNK, CHUNK)],
                                  acc_sh.at[didx.at[0]], ssem[x]).wait()

    pltpu.sync_copy(u_hbm.at[pl.ds(sid * RS, RS)], acc_sh.at[pl.ds(sid * RS, RS)])
    pltpu.sync_copy(src3_hbm.at[wid], sidx)
    pltpu.sync_copy(dst3_hbm.at[wid], didx)
    plsc.subcore_barrier()

    fire_gathers(0, 0)
    drain_gathers(0)
    fire_scatters(0, 0)
    fire_gathers(1, 1)

    def pair_body(i, _):
        for (roff, x) in ((1, 1), (2, 0)):
            r = 2 * i + roff
            drain_gathers(x)
            fire_scatters(r, x)
            drain_scatters(1 - x)
            fire_gathers(r + 1, 1 - x)
        return 0

    lax.fori_loop(0, (NROUND - 2) // 2, pair_body, 0)
    drain_gathers(1)
    fire_scatters(NROUND - 1, 1)
    drain_scatters(0)
    drain_scatters(1)
    plsc.subcore_barrier()
    pltpu.sync_copy(
        acc_sh.at[pl.ds(sid * RS, RS)],
        out_hbm.at[cid, pl.ds(sid * RS, RS)],
    )


def _project_body(x_ref, w_ref, degp_ref, u_ref, dinv_ref, winv_ref):
    deg = degp_ref[0, :, 0:1] + degp_ref[1, :, 0:1] + 1.0
    dinv = lax.rsqrt(deg)
    winv = 1.0 / deg
    z = lax.dot_general(
        x_ref[...], w_ref[...],
        dimension_numbers=(((1,), (1,)), ((), ())),
        preferred_element_type=jnp.float32,
        precision=lax.Precision.HIGHEST,
    )
    zp = jnp.concatenate([z, jnp.zeros((NPAD - N, C), jnp.float32)], axis=0)
    u_ref[...] = zp * dinv
    dinv_ref[...] = dinv
    winv_ref[...] = winv


_project = pl.pallas_call(
    _project_body,
    out_shape=(
        jax.ShapeDtypeStruct((NPAD, C), jnp.float32),
        jax.ShapeDtypeStruct((NPAD, 1), jnp.float32),
        jax.ShapeDtypeStruct((NPAD, 1), jnp.float32),
    ),
)


def _combine_body(p_ref, u_ref, winv_ref, w_ref):
    w_ref[...] = (p_ref[0] + p_ref[1] - u_ref[...]) * winv_ref[...]


_combine = pl.pallas_call(
    _combine_body,
    out_shape=jax.ShapeDtypeStruct((NPAD, C), jnp.float32),
)


def _final_body(q_ref, w_ref, dinv_ref, b_ref, o_ref):
    h = (q_ref[0] + q_ref[1] - w_ref[...]) * dinv_ref[...]
    o = h[:N] + b_ref[...]
    m = jnp.max(o, axis=1, keepdims=True)
    s = jnp.sum(jnp.exp(o - m), axis=1, keepdims=True)
    o_ref[...] = (o - m) - jnp.log(s)


_final = pl.pallas_call(
    _final_body,
    out_shape=jax.ShapeDtypeStruct((N, C), jnp.float32),
)


def kernel(x, edge_index, W, b):
    pad = N + jax.lax.rem(jnp.arange(EPAD - E, dtype=jnp.int32),
                          jnp.int32(NPAD - N))
    src3 = jnp.reshape(jnp.concatenate([edge_index[0], pad]), (NW, NCHUNK, CHUNK))
    dst3 = jnp.reshape(jnp.concatenate([edge_index[1], pad]), (NW, NCHUNK, CHUNK))
    zeros16 = jnp.zeros((NPAD, 16), jnp.float32)
    ones16 = jnp.ones((CHUNK, 16), jnp.float32)
    deg_kernel = _make_deg_kernel()
    hop_kernel = _make_hop_kernel()
    degp = deg_kernel(dst3, zeros16, ones16)
    u, dinv, winv = _project(x, W, degp)
    p = hop_kernel(u, src3, dst3)
    w = _combine(p, u, winv)
    q = hop_kernel(w, src3, dst3)
    return _final(q, w, dinv, jnp.reshape(b, (1, C)))


# speedup vs baseline: 1.1849x; 1.0062x over previous
import functools

import jax
import jax.numpy as jnp
from jax import lax
from jax.experimental import pallas as pl
from jax.experimental.pallas import tpu as pltpu
from jax.experimental.pallas import tpu_sc as plsc

N = 10000
E = 320000
F_IN = 128
C = 64

NC = 2
NS = 16
NW = NC * NS
EPT = 10240
EPAD = EPT * NW
CHUNK = 128
NCHUNK = EPT // CHUNK
KF = 4

NPAD = 10240
ROWS_PT = NPAD // NS
RS = NPAD // NS

@functools.cache
def _make_deg_kernel():
    mesh = plsc.VectorSubcoreMesh(
        core_axis_name="c", subcore_axis_name="s", num_cores=NC, num_subcores=NS
    )
    return pl.kernel(
        _deg_body,
        out_type=jax.ShapeDtypeStruct((NC, NPAD, 16), jnp.float32),
        mesh=mesh,
        compiler_params=pltpu.CompilerParams(use_tc_tiling_on_sc=False),
        scratch_types=[
            pltpu.VMEM_SHARED((NPAD, 16), jnp.float32),
            pltpu.VMEM((NCHUNK, CHUNK), jnp.int32),
            pltpu.VMEM((CHUNK, 16), jnp.float32),
            pltpu.SemaphoreType.DMA,
        ],
    )


def _deg_body(dst3_hbm, zeros_hbm, ones_hbm, out_hbm, deg_sh, didx, ones, ssem):
    cid = lax.axis_index("c")
    sid = lax.axis_index("s")
    wid = cid * NS + sid

    pltpu.sync_copy(ones_hbm, ones)
    pltpu.sync_copy(dst3_hbm.at[wid], didx)
    pltpu.sync_copy(
        zeros_hbm.at[pl.ds(sid * ROWS_PT, ROWS_PT)],
        deg_sh.at[pl.ds(sid * ROWS_PT, ROWS_PT)],
    )
    plsc.subcore_barrier()

    def fire_round(r):
        for b in range(KF):
            pltpu.async_copy(ones, deg_sh.at[didx.at[r * KF + b]], ssem,
                             add=True)

    def drain_round():
        for b in range(KF):
            pltpu.make_async_copy(ones, deg_sh.at[didx.at[0]], ssem).wait()

    fire_round(0)

    def round_body(r, _):
        fire_round(r + 1)
        drain_round()
        return 0

    lax.fori_loop(0, NCHUNK // KF - 1, round_body, 0)
    drain_round()
    plsc.subcore_barrier()
    pltpu.sync_copy(
        deg_sh.at[pl.ds(sid * ROWS_PT, ROWS_PT)],
        out_hbm.at[cid, pl.ds(sid * ROWS_PT, ROWS_PT)],
    )


@functools.cache
def _make_hop_kernel():
    mesh = plsc.VectorSubcoreMesh(
        core_axis_name="c", subcore_axis_name="s", num_cores=NC, num_subcores=NS
    )
    return pl.kernel(
        _hop_body,
        out_type=jax.ShapeDtypeStruct((NC, NPAD, C), jnp.float32),
        mesh=mesh,
        compiler_params=pltpu.CompilerParams(use_tc_tiling_on_sc=False),
        scratch_types=[
            pltpu.VMEM_SHARED((NPAD, C), jnp.float32),
            pltpu.VMEM((NCHUNK, CHUNK), jnp.int32),
            pltpu.VMEM((NCHUNK, CHUNK), jnp.int32),
            pltpu.VMEM((KF * CHUNK, C), jnp.float32),
            pltpu.VMEM((KF * CHUNK, C), jnp.float32),
            pltpu.SemaphoreType.DMA,
            pltpu.SemaphoreType.DMA,
            pltpu.SemaphoreType.DMA,
            pltpu.SemaphoreType.DMA,
        ],
    )


NROUND = NCHUNK // KF


def _hop_body(u_hbm, src3_hbm, dst3_hbm, out_hbm, acc_sh, sidx, didx,
              rows0, rows1, gsem0, gsem1, ssem0, ssem1):
    cid = lax.axis_index("c")
    sid = lax.axis_index("s")
    wid = cid * NS + sid

    rows = (rows0, rows1)
    gsem = (gsem0, gsem1)
    ssem = (ssem0, ssem1)

    def fire_gathers(r, x):
        for b in range(KF):
            pltpu.async_copy(u_hbm.at[sidx.at[r * KF + b]],
                             rows[x].at[pl.ds(b * CHUNK, CHUNK)], gsem[x])

    def drain_gathers(x):
        for b in range(KF):
            pltpu.make_async_copy(u_hbm.at[sidx.at[0]],
                                  rows[x].at[pl.ds(b * CHUNK, CHUNK)],
                                  gsem[x]).wait()

    def fire_scatters(r, x):
        for b in range(KF):
            pltpu.async_copy(rows[x].at[pl.ds(b * CHUNK, CHUNK)],
                             acc_sh.at[didx.at[r * KF + b]], ssem[x],
                             add=True)

    def drain_scatters(x):
        for b in range(KF):
            pltpu.make_async_copy(rows[x].at[pl.ds(b * CHUNK, CHUNK)],
                                  acc_sh.at[didx.at[0]], ssem[x]).wait()

    pltpu.sync_copy(u_hbm.at[pl.ds(sid * RS, RS)], acc_sh.at[pl.ds(sid * RS, RS)])
    pltpu.sync_copy(src3_hbm.at[wid], sidx)
    pltpu.sync_copy(dst3_hbm.at[wid], didx)
    plsc.subcore_barrier()

    fire_gathers(0, 0)
    drain_gathers(0)
    fire_scatters(0, 0)
    fire_gathers(1, 1)

    def pair_body(i, _):
        for (roff, x) in ((1, 1), (2, 0)):
            r = 2 * i + roff
            drain_gathers(x)
            fire_scatters(r, x)
            drain_scatters(1 - x)
            fire_gathers(r + 1, 1 - x)
        return 0

    lax.fori_loop(0, (NROUND - 2) // 2, pair_body, 0)
    drain_gathers(1)
    fire_scatters(NROUND - 1, 1)
    drain_scatters(0)
    drain_scatters(1)
    plsc.subcore_barrier()
    pltpu.sync_copy(
        acc_sh.at[pl.ds(sid * RS, RS)],
        out_hbm.at[cid, pl.ds(sid * RS, RS)],
    )


def _matmul_body(x_ref, w_ref, z_ref):
    z = lax.dot_general(
        x_ref[...], w_ref[...],
        dimension_numbers=(((1,), (1,)), ((), ())),
        preferred_element_type=jnp.float32,
        precision=lax.Precision.HIGHEST,
    )
    z_ref[...] = jnp.concatenate(
        [z, jnp.zeros((NPAD - N, C), jnp.float32)], axis=0)


_matmul = pl.pallas_call(
    _matmul_body,
    out_shape=jax.ShapeDtypeStruct((NPAD, C), jnp.float32),
)


def _scale_body(z_ref, degp_ref, u_ref, dinv_ref, winv_ref):
    deg = degp_ref[0, :, 0:1] + degp_ref[1, :, 0:1] + 1.0
    dinv = lax.rsqrt(deg)
    u_ref[...] = z_ref[...] * dinv
    dinv_ref[...] = dinv
    winv_ref[...] = 1.0 / deg


_scale = pl.pallas_call(
    _scale_body,
    out_shape=(
        jax.ShapeDtypeStruct((NPAD, C), jnp.float32),
        jax.ShapeDtypeStruct((NPAD, 1), jnp.float32),
        jax.ShapeDtypeStruct((NPAD, 1), jnp.float32),
    ),
)


def _combine_body(p_ref, u_ref, winv_ref, w_ref):
    w_ref[...] = (p_ref[0] + p_ref[1] - u_ref[...]) * winv_ref[...]


_combine = pl.pallas_call(
    _combine_body,
    out_shape=jax.ShapeDtypeStruct((NPAD, C), jnp.float32),
)


def _final_body(q_ref, w_ref, dinv_ref, b_ref, o_ref):
    h = (q_ref[0] + q_ref[1] - w_ref[...]) * dinv_ref[...]
    o = h[:N] + b_ref[...]
    m = jnp.max(o, axis=1, keepdims=True)
    s = jnp.sum(jnp.exp(o - m), axis=1, keepdims=True)
    o_ref[...] = (o - m) - jnp.log(s)


_final = pl.pallas_call(
    _final_body,
    out_shape=jax.ShapeDtypeStruct((N, C), jnp.float32),
)


def kernel(x, edge_index, W, b):
    pad = N + jax.lax.rem(jnp.arange(EPAD - E, dtype=jnp.int32),
                          jnp.int32(NPAD - N))
    src3 = jnp.reshape(jnp.concatenate([edge_index[0], pad]), (NW, NCHUNK, CHUNK))
    dst3 = jnp.reshape(jnp.concatenate([edge_index[1], pad]), (NW, NCHUNK, CHUNK))
    zeros16 = jnp.zeros((NPAD, 16), jnp.float32)
    ones16 = jnp.ones((CHUNK, 16), jnp.float32)
    deg_kernel = _make_deg_kernel()
    hop_kernel = _make_hop_kernel()
    z = _matmul(x, W)
    degp = deg_kernel(dst3, zeros16, ones16)
    u, dinv, winv = _scale(z, degp)
    p = hop_kernel(u, src3, dst3)
    w = _combine(p, u, winv)
    q = hop_kernel(w, src3, dst3)
    return _final(q, w, dinv, jnp.reshape(b, (1, C)))
